# baseline (speedup 1.0000x reference)
import functools

import jax
import jax.numpy as jnp
from jax import lax
from jax.experimental import pallas as pl
from jax.experimental.pallas import tpu as pltpu
from jax.experimental.pallas import tpu_sc as plsc

N = 10000
E = 320000
D = 128
C = 40
CP = 64
EPS = 1e-5
NC, NS = 2, 16
NW = NC * NS
K = 128
NCH = 80
EWP = NCH * K
NPAD = 10240
RPT = NPAD // NS
ZR = 128
DW_DEG = 16

_MESH = plsc.VectorSubcoreMesh(core_axis_name="c", subcore_axis_name="s")
_SC_PARAMS = pltpu.CompilerParams(use_tc_tiling_on_sc=False)


@functools.partial(
    pl.kernel,
    out_type=jax.ShapeDtypeStruct((NC, NPAD, DW_DEG), jnp.float32),
    mesh=_MESH,
    scratch_types=[
        pltpu.VMEM_SHARED((NPAD, DW_DEG), jnp.float32),
        pltpu.VMEM((NCH, 2, K), jnp.int32),
        pltpu.VMEM((K, DW_DEG), jnp.float32),
        pltpu.VMEM((ZR, DW_DEG), jnp.float32),
        pltpu.SemaphoreType.DMA,
    ],
    compiler_params=_SC_PARAMS,
)
def _sc_deg(e3_hbm, ones_hbm, zeros_hbm, out_hbm, acc, didx, ones_v, zbuf,
            sem):
    c = lax.axis_index("c")
    s = lax.axis_index("s")
    wid = c * NS + s
    pltpu.sync_copy(zeros_hbm, zbuf)
    pltpu.sync_copy(ones_hbm, ones_v)

    def zb(j, carry):
        pltpu.sync_copy(zbuf, acc.at[pl.ds(s * RPT + j * ZR, ZR)])
        return carry

    lax.fori_loop(0, RPT // ZR, zb, 0)
    pltpu.sync_copy(e3_hbm.at[wid], didx)
    plsc.subcore_barrier()

    def fire(i, carry):
        pltpu.async_copy(ones_v, acc.at[didx.at[i, 1]], sem, add=True)
        return carry

    lax.fori_loop(0, NCH, fire, 0)

    def drain(i, carry):
        pltpu.make_async_copy(ones_v, acc.at[didx.at[i, 1]], sem).wait()
        return carry

    lax.fori_loop(0, NCH, drain, 0)
    plsc.subcore_barrier()

    def outb(j, carry):
        row0 = s * RPT + j * ZR
        pltpu.sync_copy(acc.at[pl.ds(row0, ZR)], zbuf)
        pltpu.sync_copy(zbuf, out_hbm.at[c, pl.ds(row0, ZR)])
        return carry

    lax.fori_loop(0, RPT // ZR, outb, 0)


def _make_sc_agg(dw):
    @functools.partial(
        pl.kernel,
        out_type=jax.ShapeDtypeStruct((NC, NPAD, dw), jnp.float32),
        mesh=_MESH,
        scratch_types=[
            pltpu.VMEM_SHARED((NPAD, dw), jnp.float32),
            pltpu.VMEM((2, K), jnp.int32),
            pltpu.VMEM((2, K), jnp.int32),
            pltpu.VMEM((K, dw), jnp.float32),
            pltpu.VMEM((K, dw), jnp.float32),
            pltpu.SemaphoreType.DMA,
            pltpu.SemaphoreType.DMA,
            pltpu.SemaphoreType.DMA,
            pltpu.SemaphoreType.DMA,
        ],
        compiler_params=_SC_PARAMS,
    )
    def agg(xs_hbm, e3_hbm, zeros_hbm, out_hbm,
            acc, idx0, idx1, msg0, msg1, semi0, semi1, semg0, semg1):
        c = lax.axis_index("c")
        s = lax.axis_index("s")
        wid = c * NS + s
        pltpu.sync_copy(zeros_hbm, msg0)

        def zb(j, carry):
            pltpu.sync_copy(msg0, acc.at[pl.ds(s * RPT + j * ZR, ZR)])
            return carry

        lax.fori_loop(0, RPT // ZR, zb, 0)
        plsc.subcore_barrier()

        pltpu.async_copy(e3_hbm.at[wid, 0], idx0, semi0)
        pltpu.async_copy(e3_hbm.at[wid, 1], idx1, semi1)
        pltpu.make_async_copy(e3_hbm.at[wid, 0], idx0, semi0).wait()
        pltpu.async_copy(xs_hbm.at[idx0.at[0]], msg0, semg0)

        def body(i, carry):
            j0 = 2 * i
            j1 = 2 * i + 1
            pltpu.make_async_copy(e3_hbm.at[wid, j1], idx1, semi1).wait()
            pltpu.make_async_copy(xs_hbm.at[idx0.at[0]], msg0, semg0).wait()
            pltpu.async_copy(xs_hbm.at[idx1.at[0]], msg1, semg1)
            pltpu.sync_copy(msg0, acc.at[idx0.at[1]], add=True)

            @pl.when(j0 + 2 < NCH)
            def _():
                pltpu.async_copy(e3_hbm.at[wid, j0 + 2], idx0, semi0)
                pltpu.make_async_copy(e3_hbm.at[wid, j0 + 2], idx0,
                                      semi0).wait()
                pltpu.async_copy(xs_hbm.at[idx0.at[0]], msg0, semg0)

            pltpu.make_async_copy(xs_hbm.at[idx1.at[0]], msg1, semg1).wait()
            pltpu.sync_copy(msg1, acc.at[idx1.at[1]], add=True)

            @pl.when(j1 + 2 < NCH)
            def _():
                pltpu.async_copy(e3_hbm.at[wid, j1 + 2], idx1, semi1)

            return carry

        lax.fori_loop(0, NCH // 2, body, 0)
        plsc.subcore_barrier()

        def outb(j, carry):
            row0 = s * RPT + j * ZR
            pltpu.sync_copy(acc.at[pl.ds(row0, ZR)], msg0)
            pltpu.sync_copy(msg0, out_hbm.at[c, pl.ds(row0, ZR)])
            return carry

        lax.fori_loop(0, RPT // ZR, outb, 0)

    return agg


_sc_agg_d = _make_sc_agg(D)
_sc_agg_c = _make_sc_agg(CP)


def _tc_pre_body(x_ref, w1_ref, dp_ref, xs1_ref, dinv_ref):
    deg = dp_ref[0, :N, 0:1] + dp_ref[1, :N, 0:1] + 1.0
    dinv = lax.rsqrt(deg)
    xw = jnp.dot(x_ref[...], w1_ref[...], preferred_element_type=jnp.float32)
    xs1_ref[...] = xw * dinv
    dinv_ref[...] = dinv


def _tc_mid_body(s1_ref, xs1_ref, dinv_ref, b1_ref, g1_ref, be1_ref, w2_ref,
                 xs2_ref):
    dinv = dinv_ref[...]
    t = dinv * (s1_ref[0, :N, :] + s1_ref[1, :N, :] + xs1_ref[...]) + b1_ref[...]
    mean = jnp.mean(t, axis=0, keepdims=True)
    ctr = t - mean
    var = jnp.mean(ctr * ctr, axis=0, keepdims=True)
    h = g1_ref[...] * ctr * lax.rsqrt(var + EPS) + be1_ref[...]
    h = jnp.maximum(h, 0.0)
    xw2 = jnp.dot(h, w2_ref[...], preferred_element_type=jnp.float32)
    xs2_ref[...] = xw2 * dinv


def _tc_final_body(s2_ref, xs2_ref, dinv_ref, b2_ref, g2_ref, be2_ref, o_ref):
    dinv = dinv_ref[...]
    t = dinv * (s2_ref[0, :N, :] + s2_ref[1, :N, :] + xs2_ref[...]) + b2_ref[...]
    mean = jnp.mean(t, axis=0, keepdims=True)
    ctr = t - mean
    var = jnp.mean(ctr * ctr, axis=0, keepdims=True)
    o_ref[...] = g2_ref[...] * ctr * lax.rsqrt(var + EPS) + be2_ref[...]


_tc_pre = pl.pallas_call(
    _tc_pre_body,
    out_shape=[
        jax.ShapeDtypeStruct((N, D), jnp.float32),
        jax.ShapeDtypeStruct((N, 1), jnp.float32),
    ],
)

_tc_mid = pl.pallas_call(
    _tc_mid_body,
    out_shape=jax.ShapeDtypeStruct((N, CP), jnp.float32),
)

_tc_final = pl.pallas_call(
    _tc_final_body,
    out_shape=jax.ShapeDtypeStruct((N, CP), jnp.float32),
)


def kernel(x, edge_index, W1, b1, gamma1, beta1, W2, b2, gamma2, beta2):
    src = edge_index[0].astype(jnp.int32)
    dst = edge_index[1].astype(jnp.int32)
    ew = E // NW
    src_p = jnp.pad(src.reshape(NW, ew), ((0, 0), (0, EWP - ew)),
                    constant_values=0)
    dst_p = jnp.pad(dst.reshape(NW, ew), ((0, 0), (0, EWP - ew)),
                    constant_values=N)
    e3 = jnp.stack([src_p, dst_p], axis=1)
    e3 = e3.reshape(NW, 2, NCH, K).transpose(0, 2, 1, 3)

    ones16 = jnp.ones((K, DW_DEG), jnp.float32)
    zeros16 = jnp.zeros((ZR, DW_DEG), jnp.float32)
    dp = _sc_deg(e3, ones16, zeros16)

    xs1, dinv = _tc_pre(x, W1, dp)

    zeros_d = jnp.zeros((ZR, D), jnp.float32)
    s1 = _sc_agg_d(xs1, e3, zeros_d)

    W2p = jnp.pad(W2, ((0, 0), (0, CP - C)))
    xs2 = _tc_mid(s1, xs1, dinv, b1[None, :], gamma1[None, :],
                  beta1[None, :], W2p)

    zeros_c = jnp.zeros((ZR, CP), jnp.float32)
    s2 = _sc_agg_c(xs2, e3, zeros_c)

    b2p = jnp.pad(b2, (0, CP - C))[None, :]
    g2p = jnp.pad(gamma2, (0, CP - C))[None, :]
    be2p = jnp.pad(beta2, (0, CP - C))[None, :]
    out = _tc_final(s2, xs2, dinv, b2p, g2p, be2p)
    return out[:, :C]
